# baseline (speedup 1.0000x reference)
import jax
import jax.numpy as jnp
from jax.experimental import pallas as pl
from jax.experimental.pallas import tpu as pltpu

_B, _S, _D, _H, _E, _K = 32, 4, 768, 256, 8, 2
_EPS_NORM = 1e-6


def _moe_kernel(x_ref, nw_ref, gwt_ref, w1_hbm, w2_hbm, w3_hbm, out_ref,
                w1_buf, w2_buf, w3_buf, sems):
    def _copies(e):
        return (
            pltpu.make_async_copy(w1_hbm.at[e], w1_buf.at[e], sems.at[e, 0]),
            pltpu.make_async_copy(w2_hbm.at[e], w2_buf.at[e], sems.at[e, 1]),
            pltpu.make_async_copy(w3_hbm.at[e], w3_buf.at[e], sems.at[e, 2]),
        )

    for e in range(3):
        for cp in _copies(e):
            cp.start()

    x = x_ref[...]
    nw = nw_ref[...]
    xn = x * jax.lax.rsqrt(jnp.mean(x * x, axis=-1, keepdims=True) + _EPS_NORM)
    xn = xn * nw

    logits = jax.lax.dot_general(
        xn, gwt_ref[...], (((1,), (1,)), ((), ())),
        preferred_element_type=jnp.float32)
    p = jax.nn.softmax(logits, axis=-1)
    iota = jax.lax.broadcasted_iota(jnp.int32, p.shape, 1)
    m1 = jnp.max(p, axis=-1, keepdims=True)
    i1 = jnp.min(jnp.where(p >= m1, iota, _E), axis=-1, keepdims=True)
    one1 = iota == i1
    p2 = jnp.where(one1, -1.0, p)
    m2 = jnp.max(p2, axis=-1, keepdims=True)
    i2 = jnp.min(jnp.where(p2 >= m2, iota, _E), axis=-1, keepdims=True)
    one2 = iota == i2
    c = jnp.where(one1 | one2, p, 0.0) / (m1 + m2 + 1e-10)

    acc = jnp.zeros(out_ref.shape, jnp.float32)
    for e in range(_E):
        if e + 3 < _E:
            for cp in _copies(e + 3):
                cp.start()
        for cp in _copies(e):
            cp.wait()
        xnb = xn.astype(jnp.bfloat16)
        h1 = jnp.dot(xnb, w1_buf[e].astype(jnp.bfloat16),
                     preferred_element_type=jnp.float32)
        h2 = jnp.dot(xnb, w2_buf[e].astype(jnp.bfloat16),
                     preferred_element_type=jnp.float32)
        hid = (h1 * jax.lax.logistic(h1)) * h2
        oe = jnp.dot(hid.astype(jnp.bfloat16), w3_buf[e].astype(jnp.bfloat16),
                     preferred_element_type=jnp.float32)
        acc = acc + c[:, e:e + 1] * oe
    out_ref[...] = acc


def kernel(x, norm_weight, gate_w, w1, w2, w3):
    b, s, d = x.shape
    t = b * s
    x_flat = x.reshape(t, d)
    nw = norm_weight.reshape(1, d)
    out = pl.pallas_call(
        _moe_kernel,
        in_specs=[
            pl.BlockSpec((t, d), lambda: (0, 0)),
            pl.BlockSpec((1, d), lambda: (0, 0)),
            pl.BlockSpec((_E, d), lambda: (0, 0)),
            pl.BlockSpec(memory_space=pl.ANY),
            pl.BlockSpec(memory_space=pl.ANY),
            pl.BlockSpec(memory_space=pl.ANY),
        ],
        out_specs=pl.BlockSpec((t, d), lambda: (0, 0)),
        out_shape=jax.ShapeDtypeStruct((t, d), jnp.float32),
        scratch_shapes=[
            pltpu.VMEM((_E, _D, _H), jnp.float32),
            pltpu.VMEM((_E, _D, _H), jnp.float32),
            pltpu.VMEM((_E, _H, _D), jnp.float32),
            pltpu.SemaphoreType.DMA((_E, 3)),
        ],
    )(x_flat, nw, gate_w, w1, w2, w3)
    return out.reshape(b, s, d)


# speedup vs baseline: 1.0168x; 1.0168x over previous
import jax
import jax.numpy as jnp
from jax.experimental import pallas as pl
from jax.experimental.pallas import tpu as pltpu

_B, _S, _D, _H, _E, _K = 32, 4, 768, 256, 8, 2
_EPS_NORM = 1e-6


def _moe_kernel(x_ref, nw_ref, gwt_ref, w1_hbm, w2_hbm, w3_hbm, out_ref,
                w1_buf, w2_buf, w3_buf, sems):
    def _copies(e):
        return (
            pltpu.make_async_copy(w1_hbm.at[e], w1_buf.at[e], sems.at[e, 0]),
            pltpu.make_async_copy(w2_hbm.at[e], w2_buf.at[e], sems.at[e, 1]),
            pltpu.make_async_copy(w3_hbm.at[e], w3_buf.at[e], sems.at[e, 2]),
        )

    for e in range(2):
        for cp in _copies(e):
            cp.start()

    x = x_ref[...]
    nw = nw_ref[...]
    xn = x * jax.lax.rsqrt(jnp.mean(x * x, axis=-1, keepdims=True) + _EPS_NORM)
    xn = xn * nw

    logits = jax.lax.dot_general(
        xn, gwt_ref[...], (((1,), (1,)), ((), ())),
        preferred_element_type=jnp.float32)
    p = jax.nn.softmax(logits, axis=-1)
    iota = jax.lax.broadcasted_iota(jnp.int32, p.shape, 1)
    m1 = jnp.max(p, axis=-1, keepdims=True)
    i1 = jnp.min(jnp.where(p >= m1, iota, _E), axis=-1, keepdims=True)
    one1 = iota == i1
    p2 = jnp.where(one1, -1.0, p)
    m2 = jnp.max(p2, axis=-1, keepdims=True)
    i2 = jnp.min(jnp.where(p2 >= m2, iota, _E), axis=-1, keepdims=True)
    one2 = iota == i2
    c = jnp.where(one1 | one2, p, 0.0) / (m1 + m2 + 1e-10)

    acc = jnp.zeros(out_ref.shape, jnp.float32)
    for e in range(_E):
        if e + 2 < _E:
            for cp in _copies(e + 2):
                cp.start()
        for cp in _copies(e):
            cp.wait()
        xnb = xn.astype(jnp.bfloat16)
        h1 = jnp.dot(xnb, w1_buf[e].astype(jnp.bfloat16),
                     preferred_element_type=jnp.float32)
        h2 = jnp.dot(xnb, w2_buf[e].astype(jnp.bfloat16),
                     preferred_element_type=jnp.float32)
        hid = (h1 * jax.lax.logistic(h1)) * h2
        oe = jnp.dot(hid.astype(jnp.bfloat16), w3_buf[e].astype(jnp.bfloat16),
                     preferred_element_type=jnp.float32)
        acc = acc + c[:, e:e + 1] * oe
    out_ref[...] = acc


def kernel(x, norm_weight, gate_w, w1, w2, w3):
    b, s, d = x.shape
    t = b * s
    x_flat = x.reshape(t, d)
    nw = norm_weight.reshape(1, d)
    out = pl.pallas_call(
        _moe_kernel,
        in_specs=[
            pl.BlockSpec((t, d), lambda: (0, 0)),
            pl.BlockSpec((1, d), lambda: (0, 0)),
            pl.BlockSpec((_E, d), lambda: (0, 0)),
            pl.BlockSpec(memory_space=pl.ANY),
            pl.BlockSpec(memory_space=pl.ANY),
            pl.BlockSpec(memory_space=pl.ANY),
        ],
        out_specs=pl.BlockSpec((t, d), lambda: (0, 0)),
        out_shape=jax.ShapeDtypeStruct((t, d), jnp.float32),
        scratch_shapes=[
            pltpu.VMEM((_E, _D, _H), jnp.float32),
            pltpu.VMEM((_E, _D, _H), jnp.float32),
            pltpu.VMEM((_E, _H, _D), jnp.float32),
            pltpu.SemaphoreType.DMA((_E, 3)),
        ],
    )(x_flat, nw, gate_w, w1, w2, w3)
    return out.reshape(b, s, d)
